# Initial kernel scaffold; baseline (speedup 1.0000x reference)
import jax
import jax.numpy as jnp
from jax.experimental import pallas as pl


def kernel(data, segment_ids):
    raise NotImplementedError("write your pallas kernel here")


# speedup vs baseline: 2.6576x; 2.6576x over previous
import functools

import jax
import jax.numpy as jnp
import numpy as np
from jax import lax
from jax.experimental import pallas as pl
from jax.experimental.pallas import tpu as pltpu
from jax.experimental.pallas import tpu_sc as plsc

_NUM_SEGMENTS = 100000
_N = 6400000
_NC = 2
_NS = 16
_L = 16
_NW = _NC * _NS
_SEG_PER_W = _NUM_SEGMENTS // _NW
_OBUF = 3200
_CHUNK = 4096

_NEG_INF = np.float32(-np.inf)


def _take(x, idx):
    return jnp.take_along_axis(x, idx, axis=0, mode="promise_in_bounds")


def _sc_body(data_hbm, ids_hbm, starts_hbm, out_hbm, sbuf, dbuf, ibuf, obuf):
    c = lax.axis_index("c")
    s = lax.axis_index("s")
    w = c * _NS + s
    s0 = w * _SEG_PER_W

    pltpu.sync_copy(starts_hbm.at[w], sbuf)
    rng = sbuf[...]
    lo = rng[0]
    hi = rng[1]
    lo_al = lo & jnp.int32(-8)
    nchunks = (hi - lo_al + jnp.int32(_CHUNK - 1)) // jnp.int32(_CHUNK)

    neg = jnp.full((_L,), _NEG_INF, jnp.float32)

    def init_body(i, _):
        obuf[pl.ds(i * _L, _L)] = neg
        return 0

    lax.fori_loop(0, _OBUF // _L, init_body, 0)

    lanes = lax.iota(jnp.int32, _L)
    shift_idx = {d: jnp.maximum(lanes - d, 0) for d in (1, 2, 4, 8)}
    up_idx = jnp.minimum(lanes + 1, _L - 1)
    lane15 = lanes == (_L - 1)

    def vec_body(j, carry):
        cid, cmax = carry
        g = ibuf[pl.ds(j * _L, _L)]
        v = dbuf[pl.ds(j * _L, _L)]
        wv = v
        for d in (1, 2, 4, 8):
            sg = _take(g, shift_idx[d])
            sw = _take(wv, shift_idx[d])
            wv = jnp.maximum(wv, jnp.where(g == sg, sw, _NEG_INF))
        wv = jnp.maximum(wv, jnp.where(g == cid, cmax, _NEG_INF))
        up = _take(g, up_idx)
        ends = (g != up) | lane15
        lidx = g - s0
        ok = lidx.astype(jnp.uint32) < jnp.uint32(_SEG_PER_W)
        mask = ends & ok
        lclamp = jnp.where(mask, lidx, 0)
        old = plsc.load_gather(obuf, [lclamp])
        plsc.store_scatter(obuf, [lclamp], jnp.maximum(old, wv), mask=mask)
        return g[_L - 1], wv[_L - 1]

    def chunk_body(k, carry):
        base = jnp.minimum(lo_al + k * jnp.int32(_CHUNK), jnp.int32(_N - _CHUNK))
        base = pl.multiple_of(base, 8)
        pltpu.sync_copy(data_hbm.at[pl.ds(base, _CHUNK)], dbuf)
        pltpu.sync_copy(ids_hbm.at[pl.ds(base, _CHUNK)], ibuf)
        return lax.fori_loop(0, _CHUNK // _L, vec_body, carry)

    lax.fori_loop(0, nchunks, chunk_body, (jnp.int32(-1), jnp.float32(_NEG_INF)))

    pltpu.sync_copy(obuf, out_hbm.at[w])


@jax.jit
def _sc_segmax(data, ids, starts):
    mesh = plsc.VectorSubcoreMesh(
        core_axis_name="c", subcore_axis_name="s", num_cores=_NC, num_subcores=_NS
    )
    return pl.kernel(
        _sc_body,
        out_type=jax.ShapeDtypeStruct((_NW, _OBUF), jnp.float32),
        mesh=mesh,
        compiler_params=pltpu.CompilerParams(needs_layout_passes=False),
        scratch_types=[
            pltpu.VMEM((_L,), jnp.int32),
            pltpu.VMEM((_CHUNK,), jnp.float32),
            pltpu.VMEM((_CHUNK,), jnp.int32),
            pltpu.VMEM((_OBUF,), jnp.float32),
        ],
    )(data, ids, starts)


def kernel(data, segment_ids):
    ids = segment_ids.astype(jnp.int32)
    bounds = jnp.arange(0, _NUM_SEGMENTS + 1, _SEG_PER_W, dtype=jnp.int32)
    edges = jnp.searchsorted(ids, bounds, side="left").astype(jnp.int32)
    starts = jnp.zeros((_NW, _L), jnp.int32)
    starts = starts.at[:, 0].set(edges[:-1]).at[:, 1].set(edges[1:])
    out = _sc_segmax(data, ids, starts)
    return out[:, :_SEG_PER_W].reshape(_NUM_SEGMENTS)


# speedup vs baseline: 3.2193x; 1.2113x over previous
import functools

import jax
import jax.numpy as jnp
import numpy as np
from jax import lax
from jax.experimental import pallas as pl
from jax.experimental.pallas import tpu as pltpu
from jax.experimental.pallas import tpu_sc as plsc

_NUM_SEGMENTS = 100000
_N = 6400000
_NC = 2
_NS = 16
_L = 16
_NW = _NC * _NS
_SEG_PER_W = _NUM_SEGMENTS // _NW
_OBUF = 3200
_CHUNK = 8192
_UNROLL = 4

_NEG_INF = np.float32(-np.inf)


def _take(x, idx):
    return jnp.take_along_axis(x, idx, axis=0, mode="promise_in_bounds")


def _sc_body(data_hbm, ids_hbm, starts_hbm, out_hbm, sbuf, dbuf0, dbuf1,
             ibuf0, ibuf1, obuf, sd0, sd1, si0, si1):
    c = lax.axis_index("c")
    s = lax.axis_index("s")
    w = c * _NS + s
    s0 = w * _SEG_PER_W
    dbufs = (dbuf0, dbuf1)
    ibufs = (ibuf0, ibuf1)
    sd = (sd0, sd1)
    si = (si0, si1)

    pltpu.sync_copy(starts_hbm.at[w], sbuf)
    rng = sbuf[...]
    lo = rng[0]
    hi = rng[1]
    lo_al = lo & jnp.int32(-8)
    nchunks = (hi - lo_al + jnp.int32(_CHUNK - 1)) // jnp.int32(_CHUNK)

    def issue(k, b):
        base = jnp.minimum(lo_al + k * jnp.int32(_CHUNK), jnp.int32(_N - _CHUNK))
        base = pl.multiple_of(base, 8)
        pltpu.make_async_copy(data_hbm.at[pl.ds(base, _CHUNK)], dbufs[b], sd[b]).start()
        pltpu.make_async_copy(ids_hbm.at[pl.ds(base, _CHUNK)], ibufs[b], si[b]).start()

    def wait(b):
        pltpu.make_async_copy(data_hbm.at[pl.ds(0, _CHUNK)], dbufs[b], sd[b]).wait()
        pltpu.make_async_copy(ids_hbm.at[pl.ds(0, _CHUNK)], ibufs[b], si[b]).wait()

    issue(jnp.int32(0), 0)
    issue(jnp.int32(1), 1)

    neg = jnp.full((_L,), _NEG_INF, jnp.float32)

    def init_body(i, _):
        obuf[pl.ds(i * _L, _L)] = neg
        return 0

    lax.fori_loop(0, _OBUF // _L, init_body, 0)

    lanes = lax.iota(jnp.int32, _L)
    shift_idx = {d: jnp.maximum(lanes - d, 0) for d in (1, 2, 4, 8)}
    up_idx = jnp.minimum(lanes + 1, _L - 1)
    lane15 = lanes == (_L - 1)

    def one_vec(d_ref, i_ref, j, carry):
        cid, cmax = carry
        g = i_ref[pl.ds(j * _L, _L)]
        v = d_ref[pl.ds(j * _L, _L)]
        wv = v
        for d in (1, 2, 4, 8):
            sg = _take(g, shift_idx[d])
            sw = _take(wv, shift_idx[d])
            wv = jnp.maximum(wv, jnp.where(g == sg, sw, _NEG_INF))
        wv = jnp.maximum(wv, jnp.where(g == cid, cmax, _NEG_INF))
        up = _take(g, up_idx)
        ends = (g != up) | lane15
        lidx = g - s0
        ok = lidx.astype(jnp.uint32) < jnp.uint32(_SEG_PER_W)
        mask = ends & ok
        lclamp = jnp.where(mask, lidx, 0)
        old = plsc.load_gather(obuf, [lclamp])
        plsc.store_scatter(obuf, [lclamp], jnp.maximum(old, wv), mask=mask)
        return g[_L - 1], wv[_L - 1]

    def compute(b, carry):
        d_ref = dbufs[b]
        i_ref = ibufs[b]

        def grp_body(j, carry):
            for u in range(_UNROLL):
                carry = one_vec(d_ref, i_ref, j * _UNROLL + u, carry)
            return carry

        return lax.fori_loop(0, _CHUNK // _L // _UNROLL, grp_body, carry)

    def pair_body(gidx, carry):
        for b in (0, 1):
            k = gidx * 2 + b
            wait(b)
            carry = compute(b, carry)
            issue(k + 2, b)
        return carry

    gmax = (nchunks + 1) // 2
    lax.fori_loop(0, gmax, pair_body, (jnp.int32(-1), jnp.float32(_NEG_INF)))

    wait(0)
    wait(1)

    pltpu.sync_copy(obuf, out_hbm.at[w])


@jax.jit
def _sc_segmax(data, ids, starts):
    mesh = plsc.VectorSubcoreMesh(
        core_axis_name="c", subcore_axis_name="s", num_cores=_NC, num_subcores=_NS
    )
    return pl.kernel(
        _sc_body,
        out_type=jax.ShapeDtypeStruct((_NW, _OBUF), jnp.float32),
        mesh=mesh,
        compiler_params=pltpu.CompilerParams(needs_layout_passes=False),
        scratch_types=[
            pltpu.VMEM((_L,), jnp.int32),
            pltpu.VMEM((_CHUNK,), jnp.float32),
            pltpu.VMEM((_CHUNK,), jnp.float32),
            pltpu.VMEM((_CHUNK,), jnp.int32),
            pltpu.VMEM((_CHUNK,), jnp.int32),
            pltpu.VMEM((_OBUF,), jnp.float32),
            pltpu.SemaphoreType.DMA,
            pltpu.SemaphoreType.DMA,
            pltpu.SemaphoreType.DMA,
            pltpu.SemaphoreType.DMA,
        ],
    )(data, ids, starts)


def kernel(data, segment_ids):
    ids = segment_ids.astype(jnp.int32)
    bounds = jnp.arange(0, _NUM_SEGMENTS + 1, _SEG_PER_W, dtype=jnp.int32)
    edges = jnp.searchsorted(ids, bounds, side="left").astype(jnp.int32)
    starts = jnp.zeros((_NW, _L), jnp.int32)
    starts = starts.at[:, 0].set(edges[:-1]).at[:, 1].set(edges[1:])
    out = _sc_segmax(data, ids, starts)
    return out[:, :_SEG_PER_W].reshape(_NUM_SEGMENTS)
